# baseline (speedup 1.0000x reference)
import functools

import jax
import jax.numpy as jnp
from jax import lax
from jax.experimental import pallas as pl
from jax.experimental.pallas import tpu as pltpu
from jax.experimental.pallas import tpu_sc as plsc

_B, _N, _F = 64, 512, 4096
_ED = 64
_NW = 32
_BPW = _B // _NW


def _encoder_body(ent_hbm, facts_hbm, table_hbm, out_hbm,
                  ent_v, facts_v, cnt_v, ty_v, out_v,
                  sem_e, sem_f, sem_g):
    wid = lax.axis_index("s") * 2 + lax.axis_index("c")
    iota = lax.iota(jnp.int32, 16)
    ones = jnp.full((16,), 1.0, jnp.float32)

    for bb in range(_BPW):
        b = wid * _BPW + bb
        cpe = pltpu.async_copy(
            ent_hbm.at[pl.ds(b * (_N * 5), _N * 5)], ent_v, sem_e)
        cpf = pltpu.async_copy(
            facts_hbm.at[pl.ds(b * (_F * 2), _F * 2)], facts_v, sem_f)
        cpe.wait()

        gathers = []
        for j in range(4):
            @pl.loop(0, 8)
            def _types(i, j=j):
                base = j * 128 + i * 16
                rows = base + iota
                ty = plsc.load_gather(ent_v, [rows * 5 + 4])
                ty_v[j, pl.ds(i * 16, 16)] = ty.astype(jnp.int32)
                cnt_v[pl.ds(base, 16)] = jnp.zeros((16,), jnp.float32)

            gathers.append(pltpu.async_copy(
                table_hbm.at[ty_v.at[j]],
                out_v.at[pl.ds(j * 128, 128)], sem_g))

        cpf.wait()

        @pl.loop(0, _F // 16)
        def _hist(i):
            ids = plsc.load_gather(facts_v, [(i * 16 + iota) * 2 + 1])
            plsc.addupdate_scatter(cnt_v, [ids], ones)

        for g in gathers:
            g.wait()

        @pl.loop(0, _N // 16)
        def _cols(i):
            base = i * 16
            rows = base + iota
            e_base = rows * 5
            e1 = plsc.load_gather(ent_v, [e_base + 1])
            az = plsc.load_gather(ent_v, [e_base + 2])
            e3 = plsc.load_gather(ent_v, [e_base + 3])
            north = jnp.abs(az) * (1.0 / 180.0)
            east = jnp.where(az >= -90.0,
                             jnp.abs(90.0 - az),
                             90.0 + jnp.abs(az + 180.0)) * (1.0 / 180.0)
            cnt = cnt_v[pl.ds(base, 16)]
            cnt = jnp.where(rows == _N - 1, 0.0, cnt)
            ind = jnp.where(cnt > 0.0, 1.0, 0.0)
            for c, val in ((0, e1), (1, north), (2, east),
                           (3, e3), (4, cnt), (5, ind)):
                col = jnp.full((16,), c, jnp.int32)
                plsc.store_scatter(out_v, [rows, col], val)

        pltpu.sync_copy(out_v, out_hbm.at[pl.ds(b * _N, _N)])


_SCRATCH = [
    pltpu.VMEM((_N * 5,), jnp.float32),
    pltpu.VMEM((_F * 2,), jnp.int32),
    pltpu.VMEM((_N,), jnp.float32),
    pltpu.VMEM((4, 128), jnp.int32),
    pltpu.VMEM((_N, _ED), jnp.float32),
    pltpu.SemaphoreType.DMA,
    pltpu.SemaphoreType.DMA,
    pltpu.SemaphoreType.DMA,
]


def _make_encoder():
    return functools.partial(
        pl.kernel,
        out_type=jax.ShapeDtypeStruct((_B * _N, _ED), jnp.float32),
        mesh=plsc.VectorSubcoreMesh(core_axis_name="c", subcore_axis_name="s",
                                    num_cores=2, num_subcores=16),
        scratch_types=_SCRATCH,
        compiler_params=pltpu.CompilerParams(needs_layout_passes=False,
                                             use_tc_tiling_on_sc=False),
    )(_encoder_body)


def kernel(entities, facts, type_table):
    ent_flat = entities.reshape(-1)
    facts_flat = facts.astype(jnp.int32).reshape(-1)
    table_pad = jnp.concatenate(
        [jnp.zeros((type_table.shape[0], _ED - type_table.shape[1]),
                   type_table.dtype), type_table], axis=1)
    out = _make_encoder()(ent_flat, facts_flat, table_pad)
    return out.reshape(_B, _N, _ED)


# speedup vs baseline: 3.0857x; 3.0857x over previous
import functools

import jax
import jax.numpy as jnp
from jax import lax
from jax.experimental import pallas as pl
from jax.experimental.pallas import tpu as pltpu
from jax.experimental.pallas import tpu_sc as plsc

_B, _N, _F = 64, 512, 4096
_ED = 64
_NW = 32
_BPW = _B // _NW


def _encoder_body(ent_hbm, facts_hbm, table_hbm, out_hbm,
                  ent_v, ids_v, cnt_v, ty_v, out_v,
                  sem_e, sem_f, sem_g):
    wid = lax.axis_index("s") * 2 + lax.axis_index("c")
    iota = lax.iota(jnp.int32, 16)
    ones = jnp.full((16,), 1.0, jnp.float32)

    for bb in range(_BPW):
        b = wid * _BPW + bb
        cps = [pltpu.async_copy(
            ent_hbm.at[pl.ds(c * (_B * _N) + b * _N, _N)],
            ent_v.at[pl.ds((c - 1) * _N, _N)], sem_e) for c in (1, 2, 3, 4)]
        cpf = pltpu.async_copy(
            facts_hbm.at[pl.ds(b * (2 * _F), 2 * _F)], ids_v, sem_f)
        for cp in cps:
            cp.wait()

        gathers = []
        for j in range(4):
            @pl.loop(0, 8)
            def _types(i, j=j):
                base = j * 128 + i * 16
                ty = ent_v[pl.ds(3 * _N + base, 16)]
                ty_v[j, pl.ds(i * 16, 16)] = ty.astype(jnp.int32)
                cnt_v[pl.ds(base, 16)] = jnp.zeros((16,), jnp.float32)

            gathers.append(pltpu.async_copy(
                table_hbm.at[ty_v.at[j]],
                out_v.at[pl.ds(j * 128, 128)], sem_g))

        cpf.wait()

        @pl.loop(0, _F // 128)
        def _hist(j):
            base = j * 256 + 128
            for t in range(8):
                ids = ids_v[pl.ds(base + t * 16, 16)]
                plsc.addupdate_scatter(cnt_v, [ids], ones)

        for g in gathers:
            g.wait()

        @pl.loop(0, _N // 16)
        def _cols(i):
            base = i * 16
            rows = base + iota
            e1 = ent_v[pl.ds(base, 16)]
            az = ent_v[pl.ds(_N + base, 16)]
            e3 = ent_v[pl.ds(2 * _N + base, 16)]
            north = jnp.abs(az) * (1.0 / 180.0)
            east = jnp.where(az >= -90.0,
                             jnp.abs(90.0 - az),
                             90.0 + jnp.abs(az + 180.0)) * (1.0 / 180.0)
            cnt = cnt_v[pl.ds(base, 16)]
            cnt = jnp.where(rows == _N - 1, 0.0, cnt)
            ind = jnp.where(cnt > 0.0, 1.0, 0.0)
            for c, val in ((0, e1), (1, north), (2, east),
                           (3, e3), (4, cnt), (5, ind)):
                col = jnp.full((16,), c, jnp.int32)
                plsc.store_scatter(out_v, [rows, col], val)

        pltpu.sync_copy(out_v, out_hbm.at[pl.ds(b * _N, _N)])


_SCRATCH = [
    pltpu.VMEM((4 * _N,), jnp.float32),
    pltpu.VMEM((2 * _F,), jnp.int32),
    pltpu.VMEM((_N,), jnp.float32),
    pltpu.VMEM((4, 128), jnp.int32),
    pltpu.VMEM((_N, _ED), jnp.float32),
    pltpu.SemaphoreType.DMA,
    pltpu.SemaphoreType.DMA,
    pltpu.SemaphoreType.DMA,
]


def _make_encoder():
    return functools.partial(
        pl.kernel,
        out_type=jax.ShapeDtypeStruct((_B * _N, _ED), jnp.float32),
        mesh=plsc.VectorSubcoreMesh(core_axis_name="c", subcore_axis_name="s",
                                    num_cores=2, num_subcores=16),
        scratch_types=_SCRATCH,
        compiler_params=pltpu.CompilerParams(needs_layout_passes=False,
                                             use_tc_tiling_on_sc=False),
    )(_encoder_body)


def kernel(entities, facts, type_table):
    ent_flat = entities.transpose(2, 0, 1).reshape(-1)
    facts_flat = (facts.astype(jnp.int32)
                  .reshape(_B, _F // 128, 128, 2)
                  .transpose(0, 1, 3, 2)
                  .reshape(-1))
    table_pad = jnp.concatenate(
        [jnp.zeros((type_table.shape[0], _ED - type_table.shape[1]),
                   type_table.dtype), type_table], axis=1)
    out = _make_encoder()(ent_flat, facts_flat, table_pad)
    return out.reshape(_B, _N, _ED)


# speedup vs baseline: 3.1343x; 1.0158x over previous
import functools

import jax
import jax.numpy as jnp
from jax import lax
from jax.experimental import pallas as pl
from jax.experimental.pallas import tpu as pltpu
from jax.experimental.pallas import tpu_sc as plsc

_B, _N, _F = 64, 512, 4096
_ED = 64
_NW = 32
_BPW = _B // _NW


def _encoder_body(ent_hbm, facts_hbm, table_hbm, out_hbm,
                  ent_v, ids_v, cnt_v, ty_v, out_v,
                  sem_e0, sem_e1, sem_f0, sem_f1, sem_g0, sem_g1, sem_o):
    sem_e = (sem_e0, sem_e1)
    sem_f = (sem_f0, sem_f1)
    sem_g = (sem_g0, sem_g1)
    wid = lax.axis_index("s") * 2 + lax.axis_index("c")
    iota = lax.iota(jnp.int32, 16)
    ones = jnp.full((16,), 1.0, jnp.float32)

    bs = [wid * _BPW + bb for bb in range(_BPW)]

    ent_cps, facts_cps = [], []
    for k, b in enumerate(bs):
        ent_cps.append([pltpu.async_copy(
            ent_hbm.at[pl.ds(c * (_B * _N) + b * _N, _N)],
            ent_v.at[k, pl.ds((c - 1) * _N, _N)], sem_e[k])
            for c in (1, 2, 3, 4)])
        facts_cps.append(pltpu.async_copy(
            facts_hbm.at[pl.ds(b * (2 * _F), 2 * _F)], ids_v.at[k],
            sem_f[k]))

    def fire_gathers(k):
        gathers = []
        for j in range(4):
            @pl.loop(0, 8)
            def _types(i, j=j, k=k):
                base = j * 128 + i * 16
                ty = ent_v[k, pl.ds(3 * _N + base, 16)]
                ty_v[k, j, pl.ds(i * 16, 16)] = ty.astype(jnp.int32)
                cnt_v[k, 0, pl.ds(base, 16)] = jnp.zeros((16,), jnp.float32)
                cnt_v[k, 1, pl.ds(base, 16)] = jnp.zeros((16,), jnp.float32)

            gathers.append(pltpu.async_copy(
                table_hbm.at[ty_v.at[k, j]],
                out_v.at[k, pl.ds(j * 128, 128)], sem_g[k]))
        return gathers

    def histogram(k):
        @pl.loop(0, _F // 128)
        def _hist(j, k=k):
            base = j * 256 + 128
            for t in range(8):
                ids = ids_v[k, pl.ds(base + t * 16, 16)]
                plsc.addupdate_scatter(cnt_v.at[k, t % 2], [ids], ones)

    def columns(k):
        @pl.loop(0, _N // 16)
        def _cols(i, k=k):
            base = i * 16
            rows = base + iota
            e1 = ent_v[k, pl.ds(base, 16)]
            az = ent_v[k, pl.ds(_N + base, 16)]
            e3 = ent_v[k, pl.ds(2 * _N + base, 16)]
            north = jnp.abs(az) * (1.0 / 180.0)
            east = jnp.where(az >= -90.0,
                             jnp.abs(90.0 - az),
                             90.0 + jnp.abs(az + 180.0)) * (1.0 / 180.0)
            cnt = cnt_v[k, 0, pl.ds(base, 16)] + cnt_v[k, 1, pl.ds(base, 16)]
            cnt = jnp.where(rows == _N - 1, 0.0, cnt)
            ind = jnp.where(cnt > 0.0, 1.0, 0.0)
            for c, val in ((0, e1), (1, north), (2, east),
                           (3, e3), (4, cnt), (5, ind)):
                col = jnp.full((16,), c, jnp.int32)
                plsc.store_scatter(out_v.at[k], [rows, col], val)

    for cp in ent_cps[0]:
        cp.wait()
    g0 = fire_gathers(0)
    facts_cps[0].wait()
    histogram(0)
    for cp in ent_cps[1]:
        cp.wait()
    g1 = fire_gathers(1)
    for g in g0:
        g.wait()
    columns(0)
    out0 = pltpu.async_copy(
        out_v.at[0], out_hbm.at[pl.ds(bs[0] * _N, _N)], sem_o)
    facts_cps[1].wait()
    histogram(1)
    for g in g1:
        g.wait()
    columns(1)
    out1 = pltpu.async_copy(
        out_v.at[1], out_hbm.at[pl.ds(bs[1] * _N, _N)], sem_o)
    out0.wait()
    out1.wait()


_SCRATCH = [
    pltpu.VMEM((_BPW, 4 * _N), jnp.float32),
    pltpu.VMEM((_BPW, 2 * _F), jnp.int32),
    pltpu.VMEM((_BPW, 2, _N), jnp.float32),
    pltpu.VMEM((_BPW, 4, 128), jnp.int32),
    pltpu.VMEM((_BPW, _N, _ED), jnp.float32),
    pltpu.SemaphoreType.DMA,
    pltpu.SemaphoreType.DMA,
    pltpu.SemaphoreType.DMA,
    pltpu.SemaphoreType.DMA,
    pltpu.SemaphoreType.DMA,
    pltpu.SemaphoreType.DMA,
    pltpu.SemaphoreType.DMA,
]


def _make_encoder():
    return functools.partial(
        pl.kernel,
        out_type=jax.ShapeDtypeStruct((_B * _N, _ED), jnp.float32),
        mesh=plsc.VectorSubcoreMesh(core_axis_name="c", subcore_axis_name="s",
                                    num_cores=2, num_subcores=16),
        scratch_types=_SCRATCH,
        compiler_params=pltpu.CompilerParams(needs_layout_passes=False,
                                             use_tc_tiling_on_sc=False),
    )(_encoder_body)


def kernel(entities, facts, type_table):
    ent_flat = entities.transpose(2, 0, 1).reshape(-1)
    facts_flat = (facts.astype(jnp.int32)
                  .reshape(_B, _F // 128, 128, 2)
                  .transpose(0, 1, 3, 2)
                  .reshape(-1))
    table_pad = jnp.concatenate(
        [jnp.zeros((type_table.shape[0], _ED - type_table.shape[1]),
                   type_table.dtype), type_table], axis=1)
    out = _make_encoder()(ent_flat, facts_flat, table_pad)
    return out.reshape(_B, _N, _ED)


# speedup vs baseline: 4.8416x; 1.5447x over previous
import functools

import jax
import jax.numpy as jnp
from jax import lax
from jax.experimental import pallas as pl
from jax.experimental.pallas import tpu as pltpu
from jax.experimental.pallas import tpu_sc as plsc

_B, _N, _F = 64, 512, 4096
_ED = 64
_TD = 58
_NT = 100
_NW = 32
_BPW = _B // _NW


def _encoder_body(ent_hbm, facts_hbm, table_hbm, out_hbm,
                  ent_v, ids_v, cnt_v, tab_v, out_v,
                  sem_e0, sem_e1, sem_f0, sem_f1, sem_t, sem_o0, sem_o1):
    sem_e = (sem_e0, sem_e1)
    sem_f = (sem_f0, sem_f1)
    sem_o = (sem_o0, sem_o1)
    wid = lax.axis_index("s") * 2 + lax.axis_index("c")
    iota = lax.iota(jnp.int32, 16)
    ones = jnp.full((16,), 1.0, jnp.float32)

    bs = [wid * _BPW + bb for bb in range(_BPW)]

    cpt = pltpu.async_copy(table_hbm, tab_v, sem_t)
    ent_cps, facts_cps = [], []
    for k, b in enumerate(bs):
        ent_cps.append([pltpu.async_copy(
            ent_hbm.at[pl.ds(c * (_B * _N) + b * _N, _N)],
            ent_v.at[k, pl.ds((c - 1) * _N, _N)], sem_e[k])
            for c in (1, 2, 3, 4)])
        facts_cps.append(pltpu.async_copy(
            facts_hbm.at[pl.ds(b * (2 * _F), 2 * _F)], ids_v.at[k],
            sem_f[k]))

    def embed(k):
        @pl.loop(0, _N // 16)
        def _emb(i, k=k):
            base = i * 16
            ty = ent_v[k, pl.ds(3 * _N + base, 16)].astype(jnp.int32)
            for d in range(_TD):
                vals = plsc.load_gather(tab_v, [ty + d * _NT])
                out_v[k, 6 + d, pl.ds(base, 16)] = vals

    def histogram(k):
        @pl.loop(0, _F // 128)
        def _hist(j, k=k):
            base = j * 256 + 128
            for t in range(8):
                ids = ids_v[k, pl.ds(base + t * 16, 16)]
                plsc.addupdate_scatter(cnt_v.at[k, t % 2], [ids], ones)

    def columns(k):
        @pl.loop(0, _N // 16)
        def _cols(i, k=k):
            base = i * 16
            rows = base + iota
            e1 = ent_v[k, pl.ds(base, 16)]
            az = ent_v[k, pl.ds(_N + base, 16)]
            e3 = ent_v[k, pl.ds(2 * _N + base, 16)]
            north = jnp.abs(az) * (1.0 / 180.0)
            east = jnp.where(az >= -90.0,
                             jnp.abs(90.0 - az),
                             90.0 + jnp.abs(az + 180.0)) * (1.0 / 180.0)
            cnt = cnt_v[k, 0, pl.ds(base, 16)] + cnt_v[k, 1, pl.ds(base, 16)]
            cnt = jnp.where(rows == _N - 1, 0.0, cnt)
            ind = jnp.where(cnt > 0.0, 1.0, 0.0)
            out_v[k, 0, pl.ds(base, 16)] = e1
            out_v[k, 1, pl.ds(base, 16)] = north
            out_v[k, 2, pl.ds(base, 16)] = east
            out_v[k, 3, pl.ds(base, 16)] = e3
            out_v[k, 4, pl.ds(base, 16)] = cnt
            out_v[k, 5, pl.ds(base, 16)] = ind

    def zero_bins(k):
        @pl.loop(0, _N // 16)
        def _zero(i, k=k):
            cnt_v[k, 0, pl.ds(i * 16, 16)] = jnp.zeros((16,), jnp.float32)
            cnt_v[k, 1, pl.ds(i * 16, 16)] = jnp.zeros((16,), jnp.float32)

    def write_out(k):
        b = bs[k]
        return [pltpu.async_copy(
            out_v.at[k, pl.ds(ft * 8, 8), pl.ds(nt * 128, 128)],
            out_hbm.at[b * 32 + ft * 4 + nt], sem_o[k])
            for ft in range(8) for nt in range(4)]

    cpt.wait()
    zero_bins(0)
    zero_bins(1)
    for cp in ent_cps[0]:
        cp.wait()
    embed(0)
    facts_cps[0].wait()
    histogram(0)
    columns(0)
    o0 = write_out(0)
    for cp in ent_cps[1]:
        cp.wait()
    embed(1)
    facts_cps[1].wait()
    histogram(1)
    columns(1)
    o1 = write_out(1)
    for cp in o0 + o1:
        cp.wait()


_SCRATCH = [
    pltpu.VMEM((_BPW, 4 * _N), jnp.float32),
    pltpu.VMEM((_BPW, 2 * _F), jnp.int32),
    pltpu.VMEM((_BPW, 2, _N), jnp.float32),
    pltpu.VMEM((_TD * _NT,), jnp.float32),
    pltpu.VMEM((_BPW, _ED, _N), jnp.float32),
    pltpu.SemaphoreType.DMA,
    pltpu.SemaphoreType.DMA,
    pltpu.SemaphoreType.DMA,
    pltpu.SemaphoreType.DMA,
    pltpu.SemaphoreType.DMA,
    pltpu.SemaphoreType.DMA,
    pltpu.SemaphoreType.DMA,
]


def _make_encoder():
    return functools.partial(
        pl.kernel,
        out_type=jax.ShapeDtypeStruct((_B * 32, 8, 128), jnp.float32),
        mesh=plsc.VectorSubcoreMesh(core_axis_name="c", subcore_axis_name="s",
                                    num_cores=2, num_subcores=16),
        scratch_types=_SCRATCH,
        compiler_params=pltpu.CompilerParams(needs_layout_passes=False,
                                             use_tc_tiling_on_sc=False),
    )(_encoder_body)


def kernel(entities, facts, type_table):
    ent_flat = entities.transpose(2, 0, 1).reshape(-1)
    facts_flat = (facts.astype(jnp.int32)
                  .reshape(_B, _F // 128, 128, 2)
                  .transpose(0, 1, 3, 2)
                  .reshape(-1))
    table_t = type_table.T.reshape(-1)
    out = _make_encoder()(ent_flat, facts_flat, table_t)
    return (out.reshape(_B, 8, 4, 8, 128)
            .transpose(0, 2, 4, 1, 3)
            .reshape(_B, _N, _ED))
